# Initial kernel scaffold; baseline (speedup 1.0000x reference)
import jax
import jax.numpy as jnp
from jax.experimental import pallas as pl


def kernel(epoch, step, x, x_mask, x_uniq, x_uniq_mask, y, ob, emb, W, b):
    raise NotImplementedError("write your pallas kernel here")


# speedup vs baseline: 2.2902x; 2.2902x over previous
import functools

import numpy as np
import jax
import jax.numpy as jnp
from jax import lax
from jax.experimental import pallas as pl
from jax.experimental.pallas import tpu as pltpu
from jax.experimental.pallas import tpu_sc as plsc

B = 4096
L = 200
LP = 256
VOCAB = 100000
D = 128
NLAB = 12
PADLAB = 16
ATTR_LEN = (2, 4, 6)

_COLW = np.zeros((1, PADLAB), np.float32)
_o = 0
for _ln in ATTR_LEN:
    _COLW[0, _o:_o + _ln] = 1.0 / (B * _ln)
    _o += _ln


def _proj_body(emb_ref, w_ref, out_ref):
    out_ref[...] = lax.dot_general(
        emb_ref[...], w_ref[...], (((1,), (1,)), ((), ())),
        preferred_element_type=jnp.float32)


def _project(emb, wp):
    VB = 10000
    return pl.pallas_call(
        _proj_body,
        grid=(VOCAB // VB,),
        in_specs=[pl.BlockSpec((VB, D), lambda i: (i, 0)),
                  pl.BlockSpec((PADLAB, D), lambda i: (0, 0))],
        out_specs=pl.BlockSpec((VB, PADLAB), lambda i: (i, 0)),
        out_shape=jax.ShapeDtypeStruct((VOCAB, PADLAB), jnp.float32),
    )(emb, wp)


def _sc_pool(xp, p):
    info = plsc.get_sparse_core_info()
    nc, ns = info.num_cores, info.num_subcores
    nw = nc * ns
    bpw = B // nw
    nir = LP // 128
    mesh = plsc.VectorSubcoreMesh(core_axis_name="c", subcore_axis_name="s")

    def body(x_hbm, p_hbm, s_hbm, idx_v, buf, out_v, sem):
        wid = lax.axis_index("s") * nc + lax.axis_index("c")
        base = wid * bpw
        pltpu.sync_copy(x_hbm.at[pl.ds(base * nir, bpw * nir)], idx_v)

        def row(i, _):
            c0 = pltpu.async_copy(p_hbm.at[idx_v.at[nir * i]],
                                  buf.at[pl.ds(0, 128)], sem)
            c1 = pltpu.async_copy(p_hbm.at[idx_v.at[nir * i + 1]],
                                  buf.at[pl.ds(128, 128)], sem)
            c0.wait()
            c1.wait()

            def acc_body(j, accs):
                a0, a1, a2, a3 = accs
                a0 = a0 + buf[j, :]
                a1 = a1 + buf[j + 1, :]
                a2 = a2 + buf[j + 2, :]
                a3 = a3 + buf[j + 3, :]
                a0 = a0 + buf[j + 4, :]
                a1 = a1 + buf[j + 5, :]
                a2 = a2 + buf[j + 6, :]
                a3 = a3 + buf[j + 7, :]
                return a0, a1, a2, a3

            z = jnp.zeros((16,), jnp.float32)
            a0, a1, a2, a3 = lax.fori_loop(0, LP // 8,
                                           lambda k, a: acc_body(8 * k, a),
                                           (z, z, z, z))
            out_v[i, :] = (a0 + a1) + (a2 + a3)
            return 0

        lax.fori_loop(0, bpw, row, 0)
        pltpu.sync_copy(out_v, s_hbm.at[pl.ds(base, bpw)])

    f = pl.kernel(
        body,
        out_type=jax.ShapeDtypeStruct((B, PADLAB), jnp.float32),
        mesh=mesh,
        compiler_params=pltpu.CompilerParams(use_tc_tiling_on_sc=False),
        scratch_types=[
            pltpu.VMEM((bpw * nir, 128), jnp.int32),
            pltpu.VMEM((LP, PADLAB), jnp.float32),
            pltpu.VMEM((bpw, PADLAB), jnp.float32),
            pltpu.SemaphoreType.DMA,
        ],
    )
    return f(xp, p)


def _head_body(s_ref, mask_ref, y_ref, ob_ref, b_ref, colw_ref,
               logit_ref, loss_ref):
    xlen = jnp.sum(mask_ref[...], axis=1, keepdims=True)
    wc = (s_ref[...] / xlen + b_ref[...]) * ob_ref[...]
    lg = jax.nn.sigmoid(wc)
    logit_ref[...] = lg
    eps = 1e-7
    y = y_ref[...]
    t = y * jnp.log(lg + eps) + (1.0 - y) * jnp.log(1.0 - lg + eps)
    loss_ref[...] = jnp.reshape(-jnp.sum(t * colw_ref[...]), (1, 1))


def _head(s, mask_f, yp, obp, bp, colw):
    return pl.pallas_call(
        _head_body,
        out_shape=(jax.ShapeDtypeStruct((B, PADLAB), jnp.float32),
                   jax.ShapeDtypeStruct((1, 1), jnp.float32)),
    )(s, mask_f, yp, obp, bp, colw)


def kernel(epoch, step, x, x_mask, x_uniq, x_uniq_mask, y, ob, emb, W, b):
    xp = jnp.pad(x, ((0, 0), (0, LP - L))).reshape(B * LP // 128, 128)
    wp = jnp.pad(W, ((0, PADLAB - NLAB), (0, 0)))
    p = _project(emb, wp)
    s = _sc_pool(xp, p)
    mask_f = x_mask.astype(jnp.float32)
    yp = jnp.pad(y, ((0, 0), (0, PADLAB - NLAB)))
    obp = jnp.pad(ob, ((0, 0), (0, PADLAB - NLAB)))
    bp = jnp.pad(b, (0, PADLAB - NLAB)).reshape(1, PADLAB)
    colw = jnp.asarray(_COLW)
    logit16, loss = _head(s, mask_f, yp, obp, bp, colw)
    return logit16[:, :NLAB], loss.reshape(())


# speedup vs baseline: 14.0715x; 6.1443x over previous
import functools

import numpy as np
import jax
import jax.numpy as jnp
from jax import lax
from jax.experimental import pallas as pl
from jax.experimental.pallas import tpu as pltpu
from jax.experimental.pallas import tpu_sc as plsc

B = 4096
L = 200
LP = 256
VOCAB = 100000
D = 128
NLAB = 12
PADLAB = 16
ATTR_LEN = (2, 4, 6)

_COLW = np.zeros((1, PADLAB), np.float32)
_o = 0
for _ln in ATTR_LEN:
    _COLW[0, _o:_o + _ln] = 1.0 / (B * _ln)
    _o += _ln


def _proj_body(emb_ref, w_ref, out_ref):
    out_ref[...] = lax.dot_general(
        emb_ref[...], w_ref[...], (((1,), (1,)), ((), ())),
        preferred_element_type=jnp.float32)


def _project(emb, wp):
    VB = 10000
    return pl.pallas_call(
        _proj_body,
        grid=(VOCAB // VB,),
        in_specs=[pl.BlockSpec((VB, D), lambda i: (i, 0)),
                  pl.BlockSpec((PADLAB, D), lambda i: (0, 0))],
        out_specs=pl.BlockSpec((VB, PADLAB), lambda i: (i, 0)),
        out_shape=jax.ShapeDtypeStruct((VOCAB, PADLAB), jnp.float32),
    )(emb, wp)


def _sc_pool(xp, p):
    info = plsc.get_sparse_core_info()
    nc, ns = info.num_cores, info.num_subcores
    nw = nc * ns
    bpw = B // nw
    nir = LP // 128
    CH = 32
    mesh = plsc.VectorSubcoreMesh(core_axis_name="c", subcore_axis_name="s")

    def body(x_hbm, p_hbm, s_hbm, idx_v, buf, out_v, shared, sem):
        wid = lax.axis_index("s") * nc + lax.axis_index("c")
        sid = lax.axis_index("s")
        base = wid * bpw
        stripe = VOCAB // ns
        pltpu.sync_copy(p_hbm.at[pl.ds(sid * stripe, stripe)],
                        shared.at[pl.ds(sid * stripe, stripe)])
        plsc.subcore_barrier()

        def chunk(ci, _):
            pltpu.sync_copy(
                x_hbm.at[pl.ds((base + ci * CH) * nir, CH * nir)], idx_v)

            def row(i, _):
                c0 = pltpu.async_copy(shared.at[idx_v.at[nir * i]],
                                      buf.at[pl.ds(0, 128)], sem)
                c1 = pltpu.async_copy(shared.at[idx_v.at[nir * i + 1]],
                                      buf.at[pl.ds(128, 128)], sem)
                c0.wait()
                c1.wait()

                def acc_body(j, accs):
                    a0, a1, a2, a3 = accs
                    a0 = a0 + buf[j, :]
                    a1 = a1 + buf[j + 1, :]
                    a2 = a2 + buf[j + 2, :]
                    a3 = a3 + buf[j + 3, :]
                    a0 = a0 + buf[j + 4, :]
                    a1 = a1 + buf[j + 5, :]
                    a2 = a2 + buf[j + 6, :]
                    a3 = a3 + buf[j + 7, :]
                    return a0, a1, a2, a3

                z = jnp.zeros((16,), jnp.float32)
                a0, a1, a2, a3 = lax.fori_loop(0, LP // 8,
                                               lambda k, a: acc_body(8 * k, a),
                                               (z, z, z, z))
                out_v[ci * CH + i, :] = (a0 + a1) + (a2 + a3)
                return 0

            lax.fori_loop(0, CH, row, 0)
            return 0

        lax.fori_loop(0, bpw // CH, chunk, 0)
        pltpu.sync_copy(out_v, s_hbm.at[pl.ds(base, bpw)])

    f = pl.kernel(
        body,
        out_type=jax.ShapeDtypeStruct((B, PADLAB), jnp.float32),
        mesh=mesh,
        compiler_params=pltpu.CompilerParams(use_tc_tiling_on_sc=False),
        scratch_types=[
            pltpu.VMEM((CH * nir, 128), jnp.int32),
            pltpu.VMEM((LP, PADLAB), jnp.float32),
            pltpu.VMEM((bpw, PADLAB), jnp.float32),
            pltpu.VMEM_SHARED((VOCAB, PADLAB), jnp.float32),
            pltpu.SemaphoreType.DMA,
        ],
    )
    return f(xp, p)


def _head_body(s_ref, mask_ref, y_ref, ob_ref, b_ref, colw_ref,
               logit_ref, loss_ref):
    xlen = jnp.sum(mask_ref[...], axis=1, keepdims=True)
    wc = (s_ref[...] / xlen + b_ref[...]) * ob_ref[...]
    lg = jax.nn.sigmoid(wc)
    logit_ref[...] = lg
    eps = 1e-7
    y = y_ref[...]
    t = y * jnp.log(lg + eps) + (1.0 - y) * jnp.log(1.0 - lg + eps)
    loss_ref[...] = jnp.reshape(-jnp.sum(t * colw_ref[...]), (1, 1))


def _head(s, mask_f, yp, obp, bp, colw):
    return pl.pallas_call(
        _head_body,
        out_shape=(jax.ShapeDtypeStruct((B, PADLAB), jnp.float32),
                   jax.ShapeDtypeStruct((1, 1), jnp.float32)),
    )(s, mask_f, yp, obp, bp, colw)


def kernel(epoch, step, x, x_mask, x_uniq, x_uniq_mask, y, ob, emb, W, b):
    xp = jnp.pad(x, ((0, 0), (0, LP - L))).reshape(B * LP // 128, 128)
    wp = jnp.pad(W, ((0, PADLAB - NLAB), (0, 0)))
    p = _project(emb, wp)
    s = _sc_pool(xp, p)
    mask_f = x_mask.astype(jnp.float32)
    yp = jnp.pad(y, ((0, 0), (0, PADLAB - NLAB)))
    obp = jnp.pad(ob, ((0, 0), (0, PADLAB - NLAB)))
    bp = jnp.pad(b, (0, PADLAB - NLAB)).reshape(1, PADLAB)
    colw = jnp.asarray(_COLW)
    logit16, loss = _head(s, mask_f, yp, obp, bp, colw)
    return logit16[:, :NLAB], loss.reshape(())


# speedup vs baseline: 14.2392x; 1.0119x over previous
import functools

import numpy as np
import jax
import jax.numpy as jnp
from jax import lax
from jax.experimental import pallas as pl
from jax.experimental.pallas import tpu as pltpu
from jax.experimental.pallas import tpu_sc as plsc

B = 4096
L = 200
LP = 256
VOCAB = 100000
D = 128
NLAB = 12
PADLAB = 16
ATTR_LEN = (2, 4, 6)

_COLW = np.zeros((1, PADLAB), np.float32)
_o = 0
for _ln in ATTR_LEN:
    _COLW[0, _o:_o + _ln] = 1.0 / (B * _ln)
    _o += _ln


def _proj_body(emb_ref, w_ref, out_ref):
    out_ref[...] = lax.dot_general(
        emb_ref[...], w_ref[...], (((1,), (1,)), ((), ())),
        preferred_element_type=jnp.float32)


def _project(emb, wp):
    VB = 10000
    return pl.pallas_call(
        _proj_body,
        grid=(VOCAB // VB,),
        in_specs=[pl.BlockSpec((VB, D), lambda i: (i, 0)),
                  pl.BlockSpec((PADLAB, D), lambda i: (0, 0))],
        out_specs=pl.BlockSpec((VB, PADLAB), lambda i: (i, 0)),
        out_shape=jax.ShapeDtypeStruct((VOCAB, PADLAB), jnp.float32),
    )(emb, wp)


def _sc_pool(xp, p):
    info = plsc.get_sparse_core_info()
    nc, ns = info.num_cores, info.num_subcores
    nw = nc * ns
    bpw = B // nw
    nir = LP // 128
    GR = 4
    NU = GR * nir
    CH = 32
    mesh = plsc.VectorSubcoreMesh(core_axis_name="c", subcore_axis_name="s")

    def body(x_hbm, p_hbm, s_hbm, idx_v, *rest):
        bufs = rest[0:NU]
        out_v = rest[NU]
        shared = rest[NU + 1]
        sems = rest[NU + 2:NU + 2 + NU]
        wid = lax.axis_index("s") * nc + lax.axis_index("c")
        sid = lax.axis_index("s")
        base = wid * bpw
        stripe = VOCAB // ns
        pltpu.sync_copy(p_hbm.at[pl.ds(sid * stripe, stripe)],
                        shared.at[pl.ds(sid * stripe, stripe)])
        plsc.subcore_barrier()

        def acc_unit(buf, accs):
            def step(k, a):
                a0, a1, a2, a3 = a
                j = 8 * k
                a0 = a0 + buf[j, :]
                a1 = a1 + buf[j + 1, :]
                a2 = a2 + buf[j + 2, :]
                a3 = a3 + buf[j + 3, :]
                a0 = a0 + buf[j + 4, :]
                a1 = a1 + buf[j + 5, :]
                a2 = a2 + buf[j + 6, :]
                a3 = a3 + buf[j + 7, :]
                return a0, a1, a2, a3
            return lax.fori_loop(0, 128 // 8, step, accs)

        def chunk(ci, _):
            pltpu.sync_copy(
                x_hbm.at[pl.ds((base + ci * CH) * nir, CH * nir)], idx_v)

            def group(g, _):
                u0 = g * NU
                cs = [pltpu.async_copy(shared.at[idx_v.at[u0 + k]],
                                       bufs[k], sems[k])
                      for k in range(NU)]
                z = jnp.zeros((16,), jnp.float32)
                for r in range(GR):
                    cs[nir * r].wait()
                    a = acc_unit(bufs[nir * r], (z, z, z, z))
                    cs[nir * r + 1].wait()
                    a0, a1, a2, a3 = acc_unit(bufs[nir * r + 1], a)
                    out_v[ci * CH + g * GR + r, :] = (a0 + a1) + (a2 + a3)
                return 0

            lax.fori_loop(0, CH // GR, group, 0)
            return 0

        lax.fori_loop(0, bpw // CH, chunk, 0)
        pltpu.sync_copy(out_v, s_hbm.at[pl.ds(base, bpw)])

    f = pl.kernel(
        body,
        out_type=jax.ShapeDtypeStruct((B, PADLAB), jnp.float32),
        mesh=mesh,
        compiler_params=pltpu.CompilerParams(use_tc_tiling_on_sc=False),
        scratch_types=(
            [pltpu.VMEM((CH * nir, 128), jnp.int32)]
            + [pltpu.VMEM((128, PADLAB), jnp.float32) for _ in range(NU)]
            + [pltpu.VMEM((bpw, PADLAB), jnp.float32),
               pltpu.VMEM_SHARED((VOCAB, PADLAB), jnp.float32)]
            + [pltpu.SemaphoreType.DMA for _ in range(NU)]
        ),
    )
    return f(xp, p)


def _head_body(s_ref, mask_ref, y_ref, ob_ref, b_ref, colw_ref,
               logit_ref, loss_ref):
    xlen = jnp.sum(mask_ref[...], axis=1, keepdims=True)
    wc = (s_ref[...] / xlen + b_ref[...]) * ob_ref[...]
    lg = jax.nn.sigmoid(wc)
    logit_ref[...] = lg
    eps = 1e-7
    y = y_ref[...]
    t = y * jnp.log(lg + eps) + (1.0 - y) * jnp.log(1.0 - lg + eps)
    loss_ref[...] = jnp.reshape(-jnp.sum(t * colw_ref[...]), (1, 1))


def _head(s, mask_f, yp, obp, bp, colw):
    return pl.pallas_call(
        _head_body,
        out_shape=(jax.ShapeDtypeStruct((B, PADLAB), jnp.float32),
                   jax.ShapeDtypeStruct((1, 1), jnp.float32)),
    )(s, mask_f, yp, obp, bp, colw)


def kernel(epoch, step, x, x_mask, x_uniq, x_uniq_mask, y, ob, emb, W, b):
    xp = jnp.pad(x, ((0, 0), (0, LP - L))).reshape(B * LP // 128, 128)
    wp = jnp.pad(W, ((0, PADLAB - NLAB), (0, 0)))
    p = _project(emb, wp)
    s = _sc_pool(xp, p)
    mask_f = x_mask.astype(jnp.float32)
    yp = jnp.pad(y, ((0, 0), (0, PADLAB - NLAB)))
    obp = jnp.pad(ob, ((0, 0), (0, PADLAB - NLAB)))
    bp = jnp.pad(b, (0, PADLAB - NLAB)).reshape(1, PADLAB)
    colw = jnp.asarray(_COLW)
    logit16, loss = _head(s, mask_f, yp, obp, bp, colw)
    return logit16[:, :NLAB], loss.reshape(())


# speedup vs baseline: 19.2156x; 1.3495x over previous
import functools

import numpy as np
import jax
import jax.numpy as jnp
from jax import lax
from jax.experimental import pallas as pl
from jax.experimental.pallas import tpu as pltpu
from jax.experimental.pallas import tpu_sc as plsc

B = 4096
L = 200
U0 = 128
U1 = L - U0
VOCAB = 100000
D = 128
NLAB = 12
PADLAB = 16
ATTR_LEN = (2, 4, 6)

_COLW = np.zeros((1, NLAB), np.float32)
_o = 0
for _ln in ATTR_LEN:
    _COLW[0, _o:_o + _ln] = 1.0 / (B * _ln)
    _o += _ln


def _proj_body(emb_ref, w_ref, out_ref):
    out_ref[...] = lax.dot_general(
        emb_ref[...], w_ref[...], (((1,), (1,)), ((), ())),
        preferred_element_type=jnp.float32)


def _project(emb, wp):
    VB = 10000
    return pl.pallas_call(
        _proj_body,
        grid=(VOCAB // VB,),
        in_specs=[pl.BlockSpec((VB, D), lambda i: (i, 0)),
                  pl.BlockSpec((PADLAB, D), lambda i: (0, 0))],
        out_specs=pl.BlockSpec((VB, PADLAB), lambda i: (i, 0)),
        out_shape=jax.ShapeDtypeStruct((VOCAB, PADLAB), jnp.float32),
    )(emb, wp)


def _sc_pool(x, p):
    info = plsc.get_sparse_core_info()
    nc, ns = info.num_cores, info.num_subcores
    nw = nc * ns
    bpw = B // nw
    GR = 4
    NU = GR * 2
    CH = 32
    mesh = plsc.VectorSubcoreMesh(core_axis_name="c", subcore_axis_name="s")

    def body(x_hbm, p_hbm, s_hbm, idx_v, *rest):
        bufs = rest[0:NU]
        out_v = rest[NU]
        shared = rest[NU + 1]
        sems = rest[NU + 2:NU + 2 + NU]
        wid = lax.axis_index("s") * nc + lax.axis_index("c")
        sid = lax.axis_index("s")
        base = wid * bpw
        stripe = VOCAB // ns
        pltpu.sync_copy(p_hbm.at[pl.ds(sid * stripe, stripe)],
                        shared.at[pl.ds(sid * stripe, stripe)])
        plsc.subcore_barrier()

        def acc_unit(buf, n, accs):
            def step(k, a):
                a0, a1, a2, a3 = a
                j = 8 * k
                a0 = a0 + buf[j, :]
                a1 = a1 + buf[j + 1, :]
                a2 = a2 + buf[j + 2, :]
                a3 = a3 + buf[j + 3, :]
                a0 = a0 + buf[j + 4, :]
                a1 = a1 + buf[j + 5, :]
                a2 = a2 + buf[j + 6, :]
                a3 = a3 + buf[j + 7, :]
                return a0, a1, a2, a3
            return lax.fori_loop(0, n // 8, step, accs)

        def chunk(ci, _):
            pltpu.sync_copy(x_hbm.at[pl.ds(base + ci * CH, CH)], idx_v)

            def group(g, _):
                r0 = g * GR
                cs = []
                for r in range(GR):
                    cs.append(pltpu.async_copy(
                        shared.at[idx_v.at[r0 + r, pl.ds(0, U0)]],
                        bufs[2 * r], sems[2 * r]))
                    cs.append(pltpu.async_copy(
                        shared.at[idx_v.at[r0 + r, pl.ds(U0, U1)]],
                        bufs[2 * r + 1], sems[2 * r + 1]))
                z = jnp.zeros((16,), jnp.float32)
                for r in range(GR):
                    cs[2 * r].wait()
                    a = acc_unit(bufs[2 * r], U0, (z, z, z, z))
                    cs[2 * r + 1].wait()
                    a0, a1, a2, a3 = acc_unit(bufs[2 * r + 1], U1, a)
                    out_v[ci * CH + r0 + r, :] = (a0 + a1) + (a2 + a3)
                return 0

            lax.fori_loop(0, CH // GR, group, 0)
            return 0

        lax.fori_loop(0, bpw // CH, chunk, 0)
        pltpu.sync_copy(out_v, s_hbm.at[pl.ds(base, bpw)])

    f = pl.kernel(
        body,
        out_type=jax.ShapeDtypeStruct((B, PADLAB), jnp.float32),
        mesh=mesh,
        compiler_params=pltpu.CompilerParams(use_tc_tiling_on_sc=False),
        scratch_types=(
            [pltpu.VMEM((CH, L), jnp.int32)]
            + [pltpu.VMEM((U0, PADLAB), jnp.float32),
               pltpu.VMEM((U1, PADLAB), jnp.float32)] * (NU // 2)
            + [pltpu.VMEM((bpw, PADLAB), jnp.float32),
               pltpu.VMEM_SHARED((VOCAB, PADLAB), jnp.float32)]
            + [pltpu.SemaphoreType.DMA for _ in range(NU)]
        ),
    )
    return f(x, p)


def _head_body(s_ref, mask_ref, y_ref, ob_ref, b_ref, colw_ref,
               logit_ref, loss_ref):
    xlen = jnp.sum(mask_ref[...], axis=1, keepdims=True)
    s = s_ref[:, :NLAB]
    wc = (s / xlen + b_ref[...]) * ob_ref[...]
    lg = jax.nn.sigmoid(wc)
    logit_ref[...] = lg
    eps = 1e-7
    y = y_ref[...]
    t = y * jnp.log(lg + eps) + (1.0 - y) * jnp.log(1.0 - lg + eps)
    loss_ref[...] = jnp.reshape(-jnp.sum(t * colw_ref[...]), (1, 1))


def _head(s, mask_f, y, ob, b2, colw):
    return pl.pallas_call(
        _head_body,
        out_shape=(jax.ShapeDtypeStruct((B, NLAB), jnp.float32),
                   jax.ShapeDtypeStruct((1, 1), jnp.float32)),
    )(s, mask_f, y, ob, b2, colw)


def kernel(epoch, step, x, x_mask, x_uniq, x_uniq_mask, y, ob, emb, W, b):
    wp = jnp.pad(W, ((0, PADLAB - NLAB), (0, 0)))
    p = _project(emb, wp)
    s = _sc_pool(x, p)
    mask_f = x_mask.astype(jnp.float32)
    b2 = b.reshape(1, NLAB)
    colw = jnp.asarray(_COLW)
    logit, loss = _head(s, mask_f, y, ob, b2, colw)
    return logit, loss.reshape(())


# speedup vs baseline: 26.6823x; 1.3886x over previous
import functools

import numpy as np
import jax
import jax.numpy as jnp
from jax import lax
from jax.experimental import pallas as pl
from jax.experimental.pallas import tpu as pltpu
from jax.experimental.pallas import tpu_sc as plsc

B = 4096
L = 200
U0 = 128
U1 = L - U0
VOCAB = 100000
D = 128
NLAB = 12
PADLAB = 16
ATTR_LEN = (2, 4, 6)

_COLW = np.zeros((1, NLAB), np.float32)
_o = 0
for _ln in ATTR_LEN:
    _COLW[0, _o:_o + _ln] = 1.0 / (B * _ln)
    _o += _ln


_VB = 10000
_NSTEP = VOCAB // _VB
_FR = _VB * PADLAB // 128


def _proj_body(emb_ref, w_ref, out_hbm, obuf, sem):
    i = pl.program_id(0)
    slot = lax.rem(i, 2)

    @pl.when(i >= 2)
    def _wait_prev():
        pltpu.make_async_copy(
            obuf.at[slot], out_hbm.at[pl.ds((i - 2) * _FR, _FR)], sem).wait()

    emb8 = emb_ref[...].reshape(_FR, 8 * D)
    for j in range(8):
        rj = lax.dot_general(
            emb8[:, 128 * j:128 * (j + 1)], w_ref[...],
            (((1,), (1,)), ((), ())), preferred_element_type=jnp.float32)
        obuf[slot, :, PADLAB * j:PADLAB * (j + 1)] = rj
    pltpu.make_async_copy(
        obuf.at[slot], out_hbm.at[pl.ds(i * _FR, _FR)], sem).start()

    @pl.when(i == _NSTEP - 1)
    def _drain():
        pltpu.make_async_copy(
            obuf.at[1 - slot],
            out_hbm.at[pl.ds((i - 1) * _FR, _FR)], sem).wait()
        pltpu.make_async_copy(
            obuf.at[slot], out_hbm.at[pl.ds(i * _FR, _FR)], sem).wait()


def _project(emb, wp):
    p8 = pl.pallas_call(
        _proj_body,
        grid=(_NSTEP,),
        in_specs=[pl.BlockSpec((_VB, D), lambda i: (i, 0)),
                  pl.BlockSpec((PADLAB, D), lambda i: (0, 0))],
        out_specs=pl.BlockSpec(memory_space=pl.ANY),
        out_shape=jax.ShapeDtypeStruct((VOCAB // 8, 8 * PADLAB), jnp.float32),
        scratch_shapes=[pltpu.VMEM((2, _FR, 128), jnp.float32),
                        pltpu.SemaphoreType.DMA],
    )(emb, wp)
    return jnp.reshape(p8, (VOCAB, PADLAB))


def _sc_pool(x, p):
    info = plsc.get_sparse_core_info()
    nc, ns = info.num_cores, info.num_subcores
    nw = nc * ns
    bpw = B // nw
    GR = 4
    NU = GR * 2
    CH = 32
    mesh = plsc.VectorSubcoreMesh(core_axis_name="c", subcore_axis_name="s")

    def body(x_hbm, p_hbm, s_hbm, idx_v, *rest):
        bufs = rest[0:NU]
        out_v = rest[NU]
        shared = rest[NU + 1]
        sems = rest[NU + 2:NU + 2 + NU]
        wid = lax.axis_index("s") * nc + lax.axis_index("c")
        sid = lax.axis_index("s")
        base = wid * bpw
        stripe = VOCAB // ns
        pltpu.sync_copy(p_hbm.at[pl.ds(sid * stripe, stripe)],
                        shared.at[pl.ds(sid * stripe, stripe)])
        plsc.subcore_barrier()

        def acc_unit(buf, n, accs):
            def step(k, a):
                a0, a1, a2, a3 = a
                j = 8 * k
                a0 = a0 + buf[j, :]
                a1 = a1 + buf[j + 1, :]
                a2 = a2 + buf[j + 2, :]
                a3 = a3 + buf[j + 3, :]
                a0 = a0 + buf[j + 4, :]
                a1 = a1 + buf[j + 5, :]
                a2 = a2 + buf[j + 6, :]
                a3 = a3 + buf[j + 7, :]
                return a0, a1, a2, a3
            return lax.fori_loop(0, n // 8, step, accs)

        def chunk(ci, _):
            pltpu.sync_copy(x_hbm.at[pl.ds(base + ci * CH, CH)], idx_v)

            def group(g, _):
                r0 = g * GR
                cs = []
                for r in range(GR):
                    cs.append(pltpu.async_copy(
                        shared.at[idx_v.at[r0 + r, pl.ds(0, U0)]],
                        bufs[2 * r], sems[2 * r]))
                    cs.append(pltpu.async_copy(
                        shared.at[idx_v.at[r0 + r, pl.ds(U0, U1)]],
                        bufs[2 * r + 1], sems[2 * r + 1]))
                z = jnp.zeros((16,), jnp.float32)
                for r in range(GR):
                    cs[2 * r].wait()
                    a = acc_unit(bufs[2 * r], U0, (z, z, z, z))
                    cs[2 * r + 1].wait()
                    a0, a1, a2, a3 = acc_unit(bufs[2 * r + 1], U1, a)
                    out_v[ci * CH + r0 + r, :] = (a0 + a1) + (a2 + a3)
                return 0

            lax.fori_loop(0, CH // GR, group, 0)
            return 0

        lax.fori_loop(0, bpw // CH, chunk, 0)
        pltpu.sync_copy(out_v, s_hbm.at[pl.ds(base, bpw)])

    f = pl.kernel(
        body,
        out_type=jax.ShapeDtypeStruct((B, PADLAB), jnp.float32),
        mesh=mesh,
        compiler_params=pltpu.CompilerParams(use_tc_tiling_on_sc=False),
        scratch_types=(
            [pltpu.VMEM((CH, L), jnp.int32)]
            + [pltpu.VMEM((U0, PADLAB), jnp.float32),
               pltpu.VMEM((U1, PADLAB), jnp.float32)] * (NU // 2)
            + [pltpu.VMEM((bpw, PADLAB), jnp.float32),
               pltpu.VMEM_SHARED((VOCAB, PADLAB), jnp.float32)]
            + [pltpu.SemaphoreType.DMA for _ in range(NU)]
        ),
    )
    return f(x, p)


def _head_body(s_ref, mask_ref, y_ref, ob_ref, b_ref, colw_ref,
               logit_ref, loss_ref):
    xlen = jnp.sum(mask_ref[...], axis=1, keepdims=True)
    s = s_ref[:, :NLAB]
    wc = (s / xlen + b_ref[...]) * ob_ref[...]
    lg = jax.nn.sigmoid(wc)
    logit_ref[...] = lg
    eps = 1e-7
    y = y_ref[...]
    t = y * jnp.log(lg + eps) + (1.0 - y) * jnp.log(1.0 - lg + eps)
    loss_ref[...] = jnp.reshape(-jnp.sum(t * colw_ref[...]), (1, 1))


def _head(s, mask_f, y, ob, b2, colw):
    return pl.pallas_call(
        _head_body,
        out_shape=(jax.ShapeDtypeStruct((B, NLAB), jnp.float32),
                   jax.ShapeDtypeStruct((1, 1), jnp.float32)),
    )(s, mask_f, y, ob, b2, colw)


def kernel(epoch, step, x, x_mask, x_uniq, x_uniq_mask, y, ob, emb, W, b):
    wp = jnp.pad(W, ((0, PADLAB - NLAB), (0, 0)))
    p = _project(emb, wp)
    s = _sc_pool(x, p)
    mask_f = x_mask.astype(jnp.float32)
    b2 = b.reshape(1, NLAB)
    colw = jnp.asarray(_COLW)
    logit, loss = _head(s, mask_f, y, ob, b2, colw)
    return logit, loss.reshape(())
